# Initial kernel scaffold; baseline (speedup 1.0000x reference)
import jax
import jax.numpy as jnp
from jax.experimental import pallas as pl


def kernel(hidden_states, weight, bias):
    raise NotImplementedError("write your pallas kernel here")


# speedup vs baseline: 2.7722x; 2.7722x over previous
import jax
import jax.numpy as jnp
from jax.experimental import pallas as pl
from jax.experimental.pallas import tpu as pltpu

NUM_EXPERTS = 32
D_MODEL = 2880
TOP_K = 4
BT = 1024


def _router_body(hs_ref, w_ref, b_ref, scores_ref, idx_ref):
    hs = hs_ref[...]
    w = w_ref[...]
    logits32 = jax.lax.dot_general(
        hs, w, (((1,), (1,)), ((), ())), preferred_element_type=jnp.float32
    )
    s32v = logits32 + b_ref[...].astype(jnp.float32)
    v = jax.lax.bitcast_convert_type(s32v, jnp.int32)
    x = (v & jnp.int32(0x7FFFFFFF)) ^ jax.lax.shift_right_arithmetic(v, 31)
    iota = jax.lax.broadcasted_iota(jnp.int32, x.shape, 1)
    key = (x | jnp.int32(0xFFFF)) ^ iota

    int_min = jnp.int32(-2147483648)
    ms = []
    for _ in range(TOP_K):
        m = jnp.max(key, axis=1, keepdims=True)
        ms.append(m)
        key = jnp.where(key == m, int_min, key)

    idxs = [(m ^ jnp.int32(0xFFFF)) & jnp.int32(0xFFFF) for m in ms]
    vals = []
    for m in ms:
        y = (m & jnp.int32(0x7FFFFFFF)) ^ jax.lax.shift_right_arithmetic(m, 31)
        vals.append(jax.lax.bitcast_convert_type(y & jnp.int32(-65536), jnp.float32))

    vcat = jnp.concatenate(vals, axis=1)
    vmax = jnp.maximum(jnp.maximum(vals[0], vals[1]),
                       jnp.maximum(vals[2], vals[3]))
    e32 = jnp.exp(vcat - vmax)
    ssum = jnp.sum(e32, axis=1, keepdims=True)
    p32 = e32 / ssum

    idx_ref[...] = jnp.concatenate(idxs, axis=1)

    scores = jnp.zeros(x.shape, jnp.float32)
    for j in range(TOP_K):
        scores = jnp.where(iota == idxs[j], p32[:, j:j + 1], scores)
    scores_ref[...] = scores.astype(jnp.bfloat16)


def kernel(hidden_states, weight, bias):
    hs = hidden_states.reshape(-1, D_MODEL)
    n_tokens = hs.shape[0]
    grid = (n_tokens // BT,)
    bias2 = bias.reshape(1, NUM_EXPERTS)
    scores, indices = pl.pallas_call(
        _router_body,
        grid=grid,
        in_specs=[
            pl.BlockSpec((BT, D_MODEL), lambda i: (i, 0)),
            pl.BlockSpec((NUM_EXPERTS, D_MODEL), lambda i: (0, 0)),
            pl.BlockSpec((1, NUM_EXPERTS), lambda i: (0, 0)),
        ],
        out_specs=[
            pl.BlockSpec((BT, NUM_EXPERTS), lambda i: (i, 0)),
            pl.BlockSpec((BT, TOP_K), lambda i: (i, 0)),
        ],
        out_shape=[
            jax.ShapeDtypeStruct((n_tokens, NUM_EXPERTS), jnp.bfloat16),
            jax.ShapeDtypeStruct((n_tokens, TOP_K), jnp.int32),
        ],
        compiler_params=pltpu.CompilerParams(
            dimension_semantics=("arbitrary",),
        ),
    )(hs, weight, bias2)
    return scores, indices


# speedup vs baseline: 14.5933x; 5.2641x over previous
import jax
import jax.numpy as jnp
from jax.experimental import pallas as pl
from jax.experimental.pallas import tpu as pltpu

NUM_EXPERTS = 32
D_MODEL = 2880
TOP_K = 4
BT = 1024


def _router_body(hs_ref, w_ref, b_ref, scores_ref, idx_ref):
    logits32 = jax.lax.dot_general(
        w_ref[...], hs_ref[...], (((1,), (0,)), ((), ())),
        preferred_element_type=jnp.float32,
    )
    s32v = logits32 + b_ref[...].astype(jnp.float32)
    v = jax.lax.bitcast_convert_type(s32v, jnp.int32)
    x = (v & jnp.int32(0x7FFFFFFF)) ^ jax.lax.shift_right_arithmetic(v, 31)
    iota = jax.lax.broadcasted_iota(jnp.int32, x.shape, 0)
    key = (x | jnp.int32(0xFFFF)) ^ iota

    int_min = jnp.int32(-2147483648)
    ms = []
    for _ in range(TOP_K):
        m = jnp.max(key, axis=0, keepdims=True)
        ms.append(m)
        key = jnp.where(key == m, int_min, key)

    idxs = [(m ^ jnp.int32(0xFFFF)) & jnp.int32(0xFFFF) for m in ms]
    vals = []
    for m in ms:
        y = (m & jnp.int32(0x7FFFFFFF)) ^ jax.lax.shift_right_arithmetic(m, 31)
        vals.append(jax.lax.bitcast_convert_type(y & jnp.int32(-65536), jnp.float32))

    es = [jnp.exp(val - vals[0]) for val in vals]
    ssum = (es[0] + es[1]) + (es[2] + es[3])
    ps = [e / ssum for e in es]

    idx_ref[...] = jnp.concatenate(idxs, axis=0)

    scores = jnp.zeros(x.shape, jnp.float32)
    for j in range(TOP_K):
        scores = jnp.where(iota == idxs[j], ps[j], scores)
    scores_ref[...] = scores.astype(jnp.bfloat16)


def kernel(hidden_states, weight, bias):
    hs_t = hidden_states.reshape(-1, D_MODEL).T
    n_tokens = hs_t.shape[1]
    grid = (n_tokens // BT,)
    bias2 = bias.reshape(NUM_EXPERTS, 1)
    scores_t, idx_t = pl.pallas_call(
        _router_body,
        grid=grid,
        in_specs=[
            pl.BlockSpec((D_MODEL, BT), lambda i: (0, i)),
            pl.BlockSpec((NUM_EXPERTS, D_MODEL), lambda i: (0, 0)),
            pl.BlockSpec((NUM_EXPERTS, 1), lambda i: (0, 0)),
        ],
        out_specs=[
            pl.BlockSpec((NUM_EXPERTS, BT), lambda i: (0, i)),
            pl.BlockSpec((TOP_K, BT), lambda i: (0, i)),
        ],
        out_shape=[
            jax.ShapeDtypeStruct((NUM_EXPERTS, n_tokens), jnp.bfloat16),
            jax.ShapeDtypeStruct((TOP_K, n_tokens), jnp.int32),
        ],
        compiler_params=pltpu.CompilerParams(
            dimension_semantics=("arbitrary",),
        ),
    )(hs_t, weight, bias2)
    return scores_t.T, idx_t.T


# speedup vs baseline: 15.8815x; 1.0883x over previous
import jax
import jax.numpy as jnp
from jax.experimental import pallas as pl
from jax.experimental.pallas import tpu as pltpu

NUM_EXPERTS = 32
D_MODEL = 2880
TOP_K = 4
BT = 2048


def _router_body(hs_ref, w_ref, b_ref, scores_ref, idx_ref):
    logits32 = jax.lax.dot_general(
        w_ref[...], hs_ref[...], (((1,), (0,)), ((), ())),
        preferred_element_type=jnp.float32,
    )
    s32v = logits32 + b_ref[...].astype(jnp.float32)
    v = jax.lax.bitcast_convert_type(s32v, jnp.int32)
    x = (v & jnp.int32(0x7FFFFFFF)) ^ jax.lax.shift_right_arithmetic(v, 31)
    iota = jax.lax.broadcasted_iota(jnp.int32, x.shape, 0)
    key = (x | jnp.int32(0xFFFF)) ^ iota

    int_min = jnp.int32(-2147483648)
    ms = []
    for _ in range(TOP_K):
        m = jnp.max(key, axis=0, keepdims=True)
        ms.append(m)
        key = jnp.where(key == m, int_min, key)

    idxs = [(m ^ jnp.int32(0xFFFF)) & jnp.int32(0xFFFF) for m in ms]
    vals = []
    for m in ms:
        y = (m & jnp.int32(0x7FFFFFFF)) ^ jax.lax.shift_right_arithmetic(m, 31)
        vals.append(jax.lax.bitcast_convert_type(y & jnp.int32(-65536), jnp.float32))

    es = [jnp.exp(val - vals[0]) for val in vals]
    ssum = (es[0] + es[1]) + (es[2] + es[3])
    ps = [e / ssum for e in es]

    idx_ref[...] = jnp.concatenate(idxs, axis=0)

    scores = jnp.zeros(x.shape, jnp.float32)
    for j in range(TOP_K):
        scores = jnp.where(iota == idxs[j], ps[j], scores)
    scores_ref[...] = scores.astype(jnp.bfloat16)


def kernel(hidden_states, weight, bias):
    hs_t = hidden_states.reshape(-1, D_MODEL).T
    n_tokens = hs_t.shape[1]
    grid = (n_tokens // BT,)
    bias2 = bias.reshape(NUM_EXPERTS, 1)
    scores_t, idx_t = pl.pallas_call(
        _router_body,
        grid=grid,
        in_specs=[
            pl.BlockSpec((D_MODEL, BT), lambda i: (0, i)),
            pl.BlockSpec((NUM_EXPERTS, D_MODEL), lambda i: (0, 0)),
            pl.BlockSpec((NUM_EXPERTS, 1), lambda i: (0, 0)),
        ],
        out_specs=[
            pl.BlockSpec((NUM_EXPERTS, BT), lambda i: (0, i)),
            pl.BlockSpec((TOP_K, BT), lambda i: (0, i)),
        ],
        out_shape=[
            jax.ShapeDtypeStruct((NUM_EXPERTS, n_tokens), jnp.bfloat16),
            jax.ShapeDtypeStruct((TOP_K, n_tokens), jnp.int32),
        ],
        compiler_params=pltpu.CompilerParams(
            dimension_semantics=("arbitrary",),
        ),
    )(hs_t, weight, bias2)
    return scores_t.T, idx_t.T
